# baseline (speedup 1.0000x reference)
import functools

import jax
import jax.numpy as jnp
from jax import lax
from jax.experimental import pallas as pl
from jax.experimental.pallas import tpu as pltpu
from jax.experimental.pallas import tpu_sc as plsc

EPS = 1e-5


@functools.partial(jax.jit, static_argnums=(2, 3))
def _sc_gather(wte, idx, ntok, d):
    NC, NS = 2, 16
    NW = NC * NS
    b_per_w = ntok // NW
    CH = 64
    nchunk = b_per_w // CH

    mesh = plsc.VectorSubcoreMesh(core_axis_name="c", subcore_axis_name="s")

    @functools.partial(
        pl.kernel,
        mesh=mesh,
        out_type=jax.ShapeDtypeStruct((ntok, d), jnp.float32),
        scratch_types=[
            pltpu.VMEM((b_per_w,), jnp.int32),
            pltpu.VMEM((CH, d), jnp.float32),
            pltpu.VMEM((CH, d), jnp.float32),
            pltpu.SemaphoreType.DMA,
            pltpu.SemaphoreType.DMA,
            pltpu.SemaphoreType.DMA,
            pltpu.SemaphoreType.DMA,
        ],
    )
    def gather_kernel(table_hbm, idx_hbm, out_hbm, idx_v, rows0, rows1,
                      gsem0, gsem1, wsem0, wsem1):
        wid = lax.axis_index("s") * NC + lax.axis_index("c")
        base = wid * b_per_w
        rows = (rows0, rows1)
        gsems = (gsem0, gsem1)
        wsems = (wsem0, wsem1)
        pltpu.sync_copy(idx_hbm.at[pl.ds(base, b_per_w)], idx_v)

        def gather_start(c):
            return pltpu.async_copy(
                table_hbm.at[idx_v.at[pl.ds(c * CH, CH)]],
                rows[c % 2], gsems[c % 2])

        gcopies = [None] * nchunk
        wcopies = [None] * nchunk
        gcopies[0] = gather_start(0)
        for c in range(nchunk):
            if c + 1 < nchunk:
                if c >= 1:
                    wcopies[c - 1].wait()
                gcopies[c + 1] = gather_start(c + 1)
            gcopies[c].wait()
            wcopies[c] = pltpu.async_copy(
                rows[c % 2], out_hbm.at[pl.ds(base + c * CH, CH)],
                wsems[c % 2])
        wcopies[nchunk - 2].wait()
        wcopies[nchunk - 1].wait()

    return gather_kernel(wte, idx)


def _tc_body(per_t, blk, tok_ref, wpe_ref, gamma_ref, beta_ref, w_ref, b_ref,
             out_ref):
    t0 = (pl.program_id(0) % per_t) * blk
    y = tok_ref[...] + wpe_ref[pl.ds(t0, blk), :]
    mu = jnp.mean(y, axis=1, keepdims=True)
    yc = y - mu
    var = jnp.mean(yc * yc, axis=1, keepdims=True)
    z = yc * lax.rsqrt(var + EPS) * gamma_ref[...] + beta_ref[...]
    out_ref[...] = (
        lax.dot_general(z, w_ref[...], (((1,), (1,)), ((), ())),
                        preferred_element_type=jnp.float32)
        + b_ref[...]
    )


@functools.partial(jax.jit, static_argnums=(6,))
def _tc_ln_proj(tokens, wpe, gamma, beta, W, b, t_period):
    ntok, d = tokens.shape
    BLK = 512
    nblk = ntok // BLK
    per_t = t_period // BLK

    return pl.pallas_call(
        functools.partial(_tc_body, per_t, BLK),
        grid=(nblk,),
        in_specs=[
            pl.BlockSpec((BLK, d), lambda i: (i, 0)),
            pl.BlockSpec((t_period, d), lambda i: (0, 0)),
            pl.BlockSpec((1, d), lambda i: (0, 0)),
            pl.BlockSpec((1, d), lambda i: (0, 0)),
            pl.BlockSpec((d, d), lambda i: (0, 0)),
            pl.BlockSpec((1, d), lambda i: (0, 0)),
        ],
        out_specs=pl.BlockSpec((BLK, d), lambda i: (i, 0)),
        out_shape=jax.ShapeDtypeStruct((ntok, d), jnp.float32),
    )(tokens, wpe, gamma, beta, W, b)


def kernel(x, wte, wpe, gamma, beta, W, b):
    B, T = x.shape
    V, D = wte.shape
    idx = x.reshape(-1).astype(jnp.int32)
    tokens = _sc_gather(wte, idx, B * T, D)
    out = _tc_ln_proj(tokens, wpe, gamma.reshape(1, D), beta.reshape(1, D),
                      W, b.reshape(1, D), T)
    return out.reshape(B, T, D)


# speedup vs baseline: 1.0660x; 1.0660x over previous
import functools

import jax
import jax.numpy as jnp
from jax import lax
from jax.experimental import pallas as pl
from jax.experimental.pallas import tpu as pltpu
from jax.experimental.pallas import tpu_sc as plsc

EPS = 1e-5


@functools.partial(jax.jit, static_argnums=(2, 3))
def _sc_gather(wte, idx, ntok, d):
    NC, NS = 2, 16
    NW = NC * NS
    b_per_w = ntok // NW
    CH = 64
    nchunk = b_per_w // CH

    mesh = plsc.VectorSubcoreMesh(core_axis_name="c", subcore_axis_name="s")

    @functools.partial(
        pl.kernel,
        mesh=mesh,
        out_type=jax.ShapeDtypeStruct((ntok, d), jnp.float32),
        scratch_types=[
            pltpu.VMEM((b_per_w,), jnp.int32),
            pltpu.VMEM((CH, d), jnp.float32),
            pltpu.VMEM((CH, d), jnp.float32),
            pltpu.SemaphoreType.DMA,
            pltpu.SemaphoreType.DMA,
            pltpu.SemaphoreType.DMA,
            pltpu.SemaphoreType.DMA,
        ],
    )
    def gather_kernel(table_hbm, idx_hbm, out_hbm, idx_v, rows0, rows1,
                      gsem0, gsem1, wsem0, wsem1):
        wid = lax.axis_index("s") * NC + lax.axis_index("c")
        base = wid * b_per_w
        rows = (rows0, rows1)
        gsems = (gsem0, gsem1)
        wsems = (wsem0, wsem1)
        pltpu.sync_copy(idx_hbm.at[pl.ds(base, b_per_w)], idx_v)

        def gather_start(c):
            return pltpu.async_copy(
                table_hbm.at[idx_v.at[pl.ds(c * CH, CH)]],
                rows[c % 2], gsems[c % 2])

        gcopies = [None] * nchunk
        wcopies = [None] * nchunk
        gcopies[0] = gather_start(0)
        for c in range(nchunk):
            if c + 1 < nchunk:
                if c >= 1:
                    wcopies[c - 1].wait()
                gcopies[c + 1] = gather_start(c + 1)
            gcopies[c].wait()
            wcopies[c] = pltpu.async_copy(
                rows[c % 2], out_hbm.at[pl.ds(base + c * CH, CH)],
                wsems[c % 2])
        wcopies[nchunk - 2].wait()
        wcopies[nchunk - 1].wait()

    return gather_kernel(wte, idx)


def _tc_body(per_t, blk, tok_ref, wpe_ref, gamma_ref, beta_ref, w_ref, b_ref,
             out_ref):
    t0 = (pl.program_id(0) % per_t) * blk
    y = tok_ref[...] + wpe_ref[pl.ds(t0, blk), :]
    mu = jnp.mean(y, axis=1, keepdims=True)
    yc = y - mu
    var = jnp.mean(yc * yc, axis=1, keepdims=True)
    z = yc * lax.rsqrt(var + EPS) * gamma_ref[...] + beta_ref[...]
    out_ref[...] = (
        lax.dot_general(z, w_ref[...], (((1,), (1,)), ((), ())),
                        preferred_element_type=jnp.float32)
        + b_ref[...]
    )


@functools.partial(jax.jit, static_argnums=(6,))
def _tc_ln_proj(tokens, wpe, gamma, beta, W, b, t_period):
    ntok, d = tokens.shape
    BLK = 1024
    nblk = ntok // BLK
    per_t = t_period // BLK

    return pl.pallas_call(
        functools.partial(_tc_body, per_t, BLK),
        grid=(nblk,),
        in_specs=[
            pl.BlockSpec((BLK, d), lambda i: (i, 0)),
            pl.BlockSpec((t_period, d), lambda i: (0, 0)),
            pl.BlockSpec((1, d), lambda i: (0, 0)),
            pl.BlockSpec((1, d), lambda i: (0, 0)),
            pl.BlockSpec((d, d), lambda i: (0, 0)),
            pl.BlockSpec((1, d), lambda i: (0, 0)),
        ],
        out_specs=pl.BlockSpec((BLK, d), lambda i: (i, 0)),
        out_shape=jax.ShapeDtypeStruct((ntok, d), jnp.float32),
    )(tokens, wpe, gamma, beta, W, b)


def kernel(x, wte, wpe, gamma, beta, W, b):
    B, T = x.shape
    V, D = wte.shape
    idx = x.reshape(-1).astype(jnp.int32)
    tokens = _sc_gather(wte, idx, B * T, D)
    out = _tc_ln_proj(tokens, wpe, gamma.reshape(1, D), beta.reshape(1, D),
                      W, b.reshape(1, D), T)
    return out.reshape(B, T, D)


# speedup vs baseline: 1.0864x; 1.0191x over previous
import functools

import jax
import jax.numpy as jnp
from jax import lax
from jax.experimental import pallas as pl
from jax.experimental.pallas import tpu as pltpu
from jax.experimental.pallas import tpu_sc as plsc

EPS = 1e-5


@functools.partial(jax.jit, static_argnums=(2, 3))
def _sc_gather(wte, idx, ntok, d):
    NC, NS = 2, 16
    NW = NC * NS
    b_per_w = ntok // NW
    CH = 64
    nchunk = b_per_w // CH

    mesh = plsc.VectorSubcoreMesh(core_axis_name="c", subcore_axis_name="s")

    @functools.partial(
        pl.kernel,
        mesh=mesh,
        out_type=jax.ShapeDtypeStruct((ntok, d), jnp.float32),
        scratch_types=[
            pltpu.VMEM((b_per_w,), jnp.int32),
            pltpu.VMEM((CH, d), jnp.float32),
            pltpu.VMEM((CH, d), jnp.float32),
            pltpu.SemaphoreType.DMA,
            pltpu.SemaphoreType.DMA,
            pltpu.SemaphoreType.DMA,
            pltpu.SemaphoreType.DMA,
        ],
    )
    def gather_kernel(table_hbm, idx_hbm, out_hbm, idx_v, rows0, rows1,
                      gsem0, gsem1, wsem0, wsem1):
        wid = lax.axis_index("s") * NC + lax.axis_index("c")
        base = wid * b_per_w
        rows = (rows0, rows1)
        gsems = (gsem0, gsem1)
        wsems = (wsem0, wsem1)
        pltpu.sync_copy(idx_hbm.at[pl.ds(base, b_per_w)], idx_v)

        def gather_start(c):
            return pltpu.async_copy(
                table_hbm.at[idx_v.at[pl.ds(c * CH, CH)]],
                rows[c % 2], gsems[c % 2])

        gcopies = [None] * nchunk
        wcopies = [None] * nchunk
        gcopies[0] = gather_start(0)
        for c in range(nchunk):
            if c + 1 < nchunk:
                if c >= 1:
                    wcopies[c - 1].wait()
                gcopies[c + 1] = gather_start(c + 1)
            gcopies[c].wait()
            wcopies[c] = pltpu.async_copy(
                rows[c % 2], out_hbm.at[pl.ds(base + c * CH, CH)],
                wsems[c % 2])
        wcopies[nchunk - 2].wait()
        wcopies[nchunk - 1].wait()

    return gather_kernel(wte, idx)


def _tc_body(per_t, blk, tok_ref, wpe_ref, gamma_ref, beta_ref, w_ref, b_ref,
             out_ref):
    t0 = (pl.program_id(0) % per_t) * blk
    y = tok_ref[...] + wpe_ref[pl.ds(t0, blk), :]
    mu = jnp.mean(y, axis=1, keepdims=True)
    yc = y - mu
    var = jnp.mean(yc * yc, axis=1, keepdims=True)
    z = yc * lax.rsqrt(var + EPS) * gamma_ref[...] + beta_ref[...]
    out_ref[...] = (
        lax.dot_general(z, w_ref[...], (((1,), (1,)), ((), ())),
                        preferred_element_type=jnp.float32)
        + b_ref[...]
    )


@functools.partial(jax.jit, static_argnums=(6,))
def _tc_ln_proj(tokens, wpe, gamma, beta, W, b, t_period):
    ntok, d = tokens.shape
    BLK = t_period
    nblk = ntok // BLK
    per_t = t_period // BLK

    return pl.pallas_call(
        functools.partial(_tc_body, per_t, BLK),
        grid=(nblk,),
        in_specs=[
            pl.BlockSpec((BLK, d), lambda i: (i, 0)),
            pl.BlockSpec((t_period, d), lambda i: (0, 0)),
            pl.BlockSpec((d,), lambda i: (0,)),
            pl.BlockSpec((d,), lambda i: (0,)),
            pl.BlockSpec((d, d), lambda i: (0, 0)),
            pl.BlockSpec((d,), lambda i: (0,)),
        ],
        out_specs=pl.BlockSpec((BLK, d), lambda i: (i, 0)),
        out_shape=jax.ShapeDtypeStruct((ntok, d), jnp.float32),
    )(tokens, wpe, gamma, beta, W, b)


def kernel(x, wte, wpe, gamma, beta, W, b):
    B, T = x.shape
    V, D = wte.shape
    idx = x.reshape(-1).astype(jnp.int32)
    tokens = _sc_gather(wte, idx, B * T, D)
    out = _tc_ln_proj(tokens, wpe, gamma, beta, W, b, T)
    return out.reshape(B, T, D)


# speedup vs baseline: 1.0921x; 1.0053x over previous
import functools

import jax
import jax.numpy as jnp
from jax import lax
from jax.experimental import pallas as pl
from jax.experimental.pallas import tpu as pltpu
from jax.experimental.pallas import tpu_sc as plsc

EPS = 1e-5


@functools.partial(jax.jit, static_argnums=(2, 3))
def _sc_gather(wte, idx, ntok, d):
    NC, NS = 2, 16
    NW = NC * NS
    b_per_w = ntok // NW
    CH = 64
    nchunk = b_per_w // CH

    mesh = plsc.VectorSubcoreMesh(core_axis_name="c", subcore_axis_name="s")

    @functools.partial(
        pl.kernel,
        mesh=mesh,
        out_type=jax.ShapeDtypeStruct((ntok, d), jnp.float32),
        scratch_types=[
            pltpu.VMEM((b_per_w,), jnp.int32),
            pltpu.VMEM((CH, d), jnp.float32),
            pltpu.VMEM((CH, d), jnp.float32),
            pltpu.SemaphoreType.DMA,
            pltpu.SemaphoreType.DMA,
            pltpu.SemaphoreType.DMA,
            pltpu.SemaphoreType.DMA,
        ],
    )
    def gather_kernel(table_hbm, idx_hbm, out_hbm, idx_v, rows0, rows1,
                      gsem0, gsem1, wsem0, wsem1):
        wid = lax.axis_index("s") * NC + lax.axis_index("c")
        base = wid * b_per_w
        rows = (rows0, rows1)
        gsems = (gsem0, gsem1)
        wsems = (wsem0, wsem1)
        pltpu.sync_copy(idx_hbm.at[pl.ds(base, b_per_w)], idx_v)

        def gather_start(c):
            return pltpu.async_copy(
                table_hbm.at[idx_v.at[pl.ds(c * CH, CH)]],
                rows[c % 2], gsems[c % 2])

        gcopies = [None] * nchunk
        wcopies = [None] * nchunk
        gcopies[0] = gather_start(0)
        for c in range(nchunk):
            if c + 1 < nchunk:
                if c >= 1:
                    wcopies[c - 1].wait()
                gcopies[c + 1] = gather_start(c + 1)
            gcopies[c].wait()
            wcopies[c] = pltpu.async_copy(
                rows[c % 2], out_hbm.at[pl.ds(base + c * CH, CH)],
                wsems[c % 2])
        wcopies[nchunk - 2].wait()
        wcopies[nchunk - 1].wait()

    return gather_kernel(wte, idx)


def _tc_body(has_prev, tok_ref, wpe_ref, gamma_ref, beta_ref, w_ref, b_ref,
             *rest):
    out_ref = rest[-1]
    y = tok_ref[...] + wpe_ref[...]
    mu = jnp.mean(y, axis=1, keepdims=True)
    yc = y - mu
    var = jnp.mean(yc * yc, axis=1, keepdims=True)
    z = yc * lax.rsqrt(var + EPS) * gamma_ref[...] + beta_ref[...]
    out_ref[...] = (
        lax.dot_general(z, w_ref[...], (((1,), (1,)), ((), ())),
                        preferred_element_type=jnp.float32)
        + b_ref[...]
    )


def _tc_chunk_call(prev, tokens_c, wpe, gamma, beta, W, b, t_period, blk0,
                   ntok_total):
    ntok_c, d = tokens_c.shape
    BLK = t_period
    nblk = ntok_c // BLK
    has_prev = prev is not None

    in_specs = [
        pl.BlockSpec((BLK, d), lambda i: (i, 0)),
        pl.BlockSpec((t_period, d), lambda i: (0, 0)),
        pl.BlockSpec((d,), lambda i: (0,)),
        pl.BlockSpec((d,), lambda i: (0,)),
        pl.BlockSpec((d, d), lambda i: (0, 0)),
        pl.BlockSpec((d,), lambda i: (0,)),
    ]
    args = [tokens_c, wpe, gamma, beta, W, b]
    aliases = {}
    if has_prev:
        in_specs.append(pl.BlockSpec(memory_space=pltpu.MemorySpace.HBM))
        args.append(prev)
        aliases = {6: 0}

    return pl.pallas_call(
        functools.partial(_tc_body, has_prev),
        grid=(nblk,),
        in_specs=in_specs,
        out_specs=pl.BlockSpec((BLK, d), lambda i, b0=blk0: (b0 + i, 0)),
        out_shape=jax.ShapeDtypeStruct((ntok_total, d), jnp.float32),
        input_output_aliases=aliases,
    )(*args)


def kernel(x, wte, wpe, gamma, beta, W, b):
    B, T = x.shape
    V, D = wte.shape
    NCH = 2
    ntok = B * T
    ck = ntok // NCH
    idx = x.reshape(-1).astype(jnp.int32)
    toks = [_sc_gather(wte, lax.slice(idx, (c * ck,), ((c + 1) * ck,)), ck, D)
            for c in range(NCH)]
    out = None
    for c in range(NCH):
        out = _tc_chunk_call(out, toks[c], wpe, gamma, beta, W, b,
                             T, c * (ck // T), ntok)
    return out.reshape(B, T, D)
